# Initial kernel scaffold; baseline (speedup 1.0000x reference)
import jax
import jax.numpy as jnp
from jax.experimental import pallas as pl


def kernel(x, pos_emb):
    raise NotImplementedError("write your pallas kernel here")


# speedup vs baseline: 2.9029x; 2.9029x over previous
import jax
import jax.numpy as jnp
from jax.experimental import pallas as pl


def _add_block(x_ref, pe_ref, o_ref):
    o_ref[...] = x_ref[...] + pe_ref[...]


def kernel(x, pos_emb):
    b, t, c = x.shape
    bt = 512
    grid = (t // bt, b)
    return pl.pallas_call(
        _add_block,
        grid=grid,
        in_specs=[
            pl.BlockSpec((1, bt, c), lambda i, j: (j, i, 0)),
            pl.BlockSpec((bt, c), lambda i, j: (i, 0)),
        ],
        out_specs=pl.BlockSpec((1, bt, c), lambda i, j: (j, i, 0)),
        out_shape=jax.ShapeDtypeStruct((b, t, c), x.dtype),
    )(x, pos_emb)


# speedup vs baseline: 3.2442x; 1.1176x over previous
import jax
import jax.numpy as jnp
from jax.experimental import pallas as pl


def _add_block(x_ref, pe_ref, o_ref):
    o_ref[...] = x_ref[...] + pe_ref[...]


def kernel(x, pos_emb):
    b, t, c = x.shape
    bt = 1024
    grid = (t // bt, b)
    return pl.pallas_call(
        _add_block,
        grid=grid,
        in_specs=[
            pl.BlockSpec((1, bt, c), lambda i, j: (j, i, 0)),
            pl.BlockSpec((bt, c), lambda i, j: (i, 0)),
        ],
        out_specs=pl.BlockSpec((1, bt, c), lambda i, j: (j, i, 0)),
        out_shape=jax.ShapeDtypeStruct((b, t, c), x.dtype),
    )(x, pos_emb)


# speedup vs baseline: 3.4668x; 1.0686x over previous
import jax
import jax.numpy as jnp
from jax.experimental import pallas as pl


def _add_block(x_ref, pe_ref, o_ref):
    o_ref[...] = x_ref[...] + pe_ref[...]


def kernel(x, pos_emb):
    b, t, c = x.shape
    bt = 2048
    grid = (t // bt, b)
    return pl.pallas_call(
        _add_block,
        grid=grid,
        in_specs=[
            pl.BlockSpec((1, bt, c), lambda i, j: (j, i, 0)),
            pl.BlockSpec((bt, c), lambda i, j: (i, 0)),
        ],
        out_specs=pl.BlockSpec((1, bt, c), lambda i, j: (j, i, 0)),
        out_shape=jax.ShapeDtypeStruct((b, t, c), x.dtype),
    )(x, pos_emb)
